# baseline (speedup 1.0000x reference)
import functools

import jax
import jax.numpy as jnp
from jax import lax
from jax.experimental import pallas as pl
from jax.experimental.pallas import tpu as pltpu
from jax.experimental.pallas import tpu_sc as plsc

_N, _C, _H, _W = 2, 490, 50, 50
_PH, _PW = 7, 7
_NBINS = _PH * _PW
_COUT = _C // _NBINS
_SCALE = 0.0625
_GRID = 2
_TERMS = _NBINS * _GRID * _GRID * 4
_NROIS = 512
_VROWS = _N * _NBINS * _H * _W
_LANES = 16

_NCORES, _NSUBCORES = 2, 16
_NTILES = _NCORES * _NSUBCORES
_ROIS_PER_TILE = _NROIS // _NTILES
_CHUNK = 112
_NCHUNKS = _TERMS // _CHUNK


_HW = _H * _W
_NPAIRS = _N * _NBINS
_KSTEPS = _HW // _LANES
_KREM = _HW - _KSTEPS * _LANES


_SUB = 40
_ROWW = _N * _NBINS
_PIECE_IN = _SUB * _N * _C
_PIECE_OUT = _SUB * _ROWW * _LANES


def _table_body(inp_hbm, out_hbm, in_v, out_v, sem):
    wid = lax.axis_index("s") * _NCORES + lax.axis_index("c")
    s_lo = wid * 78 + jnp.minimum(wid, 4)
    cnt = 78 + (wid < 4).astype(jnp.int32)
    lanes = lax.iota(jnp.int32, _LANES)
    cmask = lanes < _COUT
    cvec = lanes * _NBINS

    def per_piece(s0):
        pltpu.sync_copy(inp_hbm.at[pl.ds(s0, _SUB)], in_v)

        def per_pos(si, carry):
            siv = lanes * 0 + si
            obase = si * (_ROWW * _LANES)
            for n in range(_N):
                nv = lanes * 0 + n
                for b in range(_NBINS):
                    v = plsc.load_gather(in_v, [siv, nv, cvec + b], mask=cmask)
                    out_v[pl.ds(obase + (n * _NBINS + b) * _LANES, _LANES)] = v
            return carry

        lax.fori_loop(0, _SUB, per_pos, 0)
        pltpu.sync_copy(out_v, out_hbm.at[pl.ds(s0 * (_ROWW * _LANES), _PIECE_OUT)])

    per_piece(s_lo)
    per_piece(s_lo + cnt - _SUB)


@functools.cache
def _table_builder():
    return pl.kernel(
        _table_body,
        out_type=jax.ShapeDtypeStruct((_VROWS * _LANES,), jnp.float32),
        mesh=plsc.VectorSubcoreMesh(
            core_axis_name="c", subcore_axis_name="s",
            num_cores=_NCORES, num_subcores=_NSUBCORES,
        ),
        scratch_types=[
            pltpu.VMEM((_SUB, _N, _C), jnp.float32),
            pltpu.VMEM((_PIECE_OUT,), jnp.float32),
            pltpu.SemaphoreType.DMA,
        ],
        compiler_params=pltpu.CompilerParams(
            use_tc_tiling_on_sc=False, needs_layout_passes=False
        ),
    )


def _build_table(input):
    x = input.transpose(2, 3, 0, 1).reshape(_HW, _N, _C)
    return _table_builder()(x).reshape(_VROWS, _LANES)


def _terms_kernel(rois_ref, idx_ref, w_ref):
    shp = (_NROIS, _TERMS)
    t = lax.broadcasted_iota(jnp.int32, shp, 1)
    b = t // 16
    j = t - 16 * b
    ph = b // _PW
    pw = b - _PW * ph
    iy = j // 8
    ix = (j - 8 * iy) // 4
    c = j - 8 * iy - 4 * ix

    n = rois_ref[:, 0:1].astype(jnp.int32)
    sw = rois_ref[:, 1:2] * _SCALE - 0.5
    sh = rois_ref[:, 2:3] * _SCALE - 0.5
    ew = rois_ref[:, 3:4] * _SCALE - 0.5
    eh = rois_ref[:, 4:5] * _SCALE - 0.5
    bh = (eh - sh) * (1.0 / _PH)
    bw = (ew - sw) * (1.0 / _PW)

    y = sh + ph.astype(jnp.float32) * bh + (iy.astype(jnp.float32) + 0.5) * bh * (1.0 / _GRID)
    x = sw + pw.astype(jnp.float32) * bw + (ix.astype(jnp.float32) + 0.5) * bw * (1.0 / _GRID)
    valid = (y >= -1.0) & (y <= float(_H)) & (x >= -1.0) & (x <= float(_W))

    yc = jnp.maximum(y, 0.0)
    y_low = jnp.floor(yc).astype(jnp.int32)
    y_edge = y_low >= _H - 1
    y_high = jnp.where(y_edge, _H - 1, y_low + 1)
    y_low = jnp.where(y_edge, _H - 1, y_low)
    yc = jnp.where(y_edge, y_low.astype(jnp.float32), yc)
    ly = yc - y_low.astype(jnp.float32)
    hy = 1.0 - ly

    xc = jnp.maximum(x, 0.0)
    x_low = jnp.floor(xc).astype(jnp.int32)
    x_edge = x_low >= _W - 1
    x_high = jnp.where(x_edge, _W - 1, x_low + 1)
    x_low = jnp.where(x_edge, _W - 1, x_low)
    xc = jnp.where(x_edge, x_low.astype(jnp.float32), xc)
    lx = xc - x_low.astype(jnp.float32)
    hx = 1.0 - lx

    yp = jnp.where(c >= 2, y_high, y_low)
    xp = jnp.where(c % 2 == 1, x_high, x_low)
    wy = jnp.where(c >= 2, ly, hy)
    wx = jnp.where(c % 2 == 1, lx, hx)
    w = jnp.where(valid, wy * wx * (1.0 / (_GRID * _GRID)), 0.0)

    row = (yp * _W + xp) * (_N * _NBINS) + n * _NBINS + b
    row = jnp.clip(row, 0, _VROWS - 1)
    idx_ref[...] = row
    w_ref[...] = w


def _compute_terms(rois):
    return pl.pallas_call(
        _terms_kernel,
        out_shape=(
            jax.ShapeDtypeStruct((_NROIS, _TERMS), jnp.int32),
            jax.ShapeDtypeStruct((_NROIS, _TERMS), jnp.float32),
        ),
    )(rois)


def _sc_body(table_hbm, idx_hbm, w_hbm, out_hbm, idx_v, w_v, g_v, out_v, sem):
    wid = lax.axis_index("s") * _NCORES + lax.axis_index("c")
    base = wid * _ROIS_PER_TILE
    pltpu.sync_copy(idx_hbm.at[pl.ds(base, _ROIS_PER_TILE)], idx_v)
    pltpu.sync_copy(w_hbm.at[pl.ds(base, _ROIS_PER_TILE)], w_v)
    lanes = lax.iota(jnp.int32, _LANES)
    cmask = lanes < _COUT
    cidx = lanes * _NBINS

    def per_roi(r, carry):
        copies = [
            pltpu.async_copy(
                table_hbm.at[idx_v.at[r, j]],
                g_v.at[pl.ds(j * _CHUNK, _CHUNK)],
                sem,
            )
            for j in range(_NCHUNKS)
        ]
        for cp in copies:
            cp.wait()
        for b in range(_NBINS):
            wvec = w_v[r, pl.ds(b * 16, 16)]
            acc = wvec[0] * g_v[b * 16, :]
            for j in range(1, 16):
                acc = acc + wvec[j] * g_v[b * 16 + j, :]
            plsc.store_scatter(out_v, [cidx + b], acc, mask=cmask)
        pltpu.sync_copy(out_v, out_hbm.at[base + r])
        return carry

    lax.fori_loop(0, _ROIS_PER_TILE, per_roi, 0)


@functools.cache
def _sc_gather():
    return pl.kernel(
        _sc_body,
        out_type=jax.ShapeDtypeStruct((_NROIS, _C), jnp.float32),
        mesh=plsc.VectorSubcoreMesh(
            core_axis_name="c", subcore_axis_name="s",
            num_cores=_NCORES, num_subcores=_NSUBCORES,
        ),
        scratch_types=[
            pltpu.VMEM((_ROIS_PER_TILE, _NCHUNKS, _CHUNK), jnp.int32),
            pltpu.VMEM((_ROIS_PER_TILE, _TERMS), jnp.float32),
            pltpu.VMEM((_TERMS, _LANES), jnp.float32),
            pltpu.VMEM((_C,), jnp.float32),
            pltpu.SemaphoreType.DMA,
        ],
        compiler_params=pltpu.CompilerParams(
            use_tc_tiling_on_sc=False, needs_layout_passes=False
        ),
    )


def kernel(input, rois):
    table = _build_table(input)
    idx, w = _compute_terms(rois)
    idx = idx.reshape(_NROIS, _NCHUNKS, _CHUNK)

    out = _sc_gather()(table, idx, w)
    return out.reshape(_NROIS, _COUT, _PH, _PW)


# speedup vs baseline: 1.4091x; 1.4091x over previous
import functools

import jax
import jax.numpy as jnp
from jax import lax
from jax.experimental import pallas as pl
from jax.experimental.pallas import tpu as pltpu
from jax.experimental.pallas import tpu_sc as plsc

_N, _C, _H, _W = 2, 490, 50, 50
_PH, _PW = 7, 7
_NBINS = _PH * _PW
_COUT = _C // _NBINS
_SCALE = 0.0625
_GRID = 2
_TERMS = _NBINS * _GRID * _GRID * 4
_NROIS = 512
_VROWS = _N * _NBINS * _H * _W
_LANES = 16

_NCORES, _NSUBCORES = 2, 16
_NTILES = _NCORES * _NSUBCORES
_ROIS_PER_TILE = _NROIS // _NTILES
_CHUNK = 112
_NCHUNKS = _TERMS // _CHUNK


_HW = _H * _W
_NPAIRS = _N * _NBINS
_KSTEPS = _HW // _LANES
_KREM = _HW - _KSTEPS * _LANES


_SUB = 40
_ROWW = _N * _NBINS
_PIECE_IN = _SUB * _N * _C
_PIECE_OUT = _SUB * _ROWW * _LANES


def _table_body(inp_hbm, out_hbm, in_v, out_v, idx_v, sem):
    wid = lax.axis_index("s") * _NCORES + lax.axis_index("c")
    s_lo = wid * 78 + jnp.minimum(wid, 4)
    cnt = 78 + (wid < 4).astype(jnp.int32)
    lanes = lax.iota(jnp.int32, _LANES)
    cmask = lanes < _COUT
    cvec = lanes * _NBINS
    out_hbm2 = out_hbm

    def per_piece(s0):
        pltpu.sync_copy(inp_hbm.at[pl.ds(s0, _SUB)], in_v)

        def per_pos(si, carry):
            siv = lanes * 0 + si
            for n in range(_N):
                nv = lanes * 0 + n
                for b in range(_NBINS):
                    v = plsc.load_gather(in_v, [siv, nv, cvec + b], mask=cmask)
                    out_v[si * _ROWW + n * _NBINS + b, :] = v
            return carry

        lax.fori_loop(0, _SUB, per_pos, 0)

        def idx_chunk(t, carry):
            kv = t * _LANES + lanes
            siv = kv // _ROWW
            gv = kv - siv * _ROWW
            j = t // 7
            q = t - j * 7
            idx_v[j, 0, pl.ds(q * _LANES, _LANES)] = gv * _HW + (s0 + siv)
            return carry

        lax.fori_loop(0, _SUB * _ROWW // _LANES, idx_chunk, 0)
        copies = [
            pltpu.async_copy(
                out_v.at[pl.ds(j * _CHUNK, _CHUNK)],
                out_hbm2.at[idx_v.at[j, 0]],
                sem,
            )
            for j in range(_SUB * _ROWW // _CHUNK)
        ]
        for cp in copies:
            cp.wait()

    per_piece(s_lo)
    per_piece(s_lo + cnt - _SUB)


@functools.cache
def _table_builder():
    return pl.kernel(
        _table_body,
        out_type=jax.ShapeDtypeStruct((_VROWS, _LANES), jnp.float32),
        mesh=plsc.VectorSubcoreMesh(
            core_axis_name="c", subcore_axis_name="s",
            num_cores=_NCORES, num_subcores=_NSUBCORES,
        ),
        scratch_types=[
            pltpu.VMEM((_SUB, _N, _C), jnp.float32),
            pltpu.VMEM((_SUB * _ROWW, _LANES), jnp.float32),
            pltpu.VMEM((_SUB * _ROWW // _CHUNK, 1, _CHUNK), jnp.int32),
            pltpu.SemaphoreType.DMA,
        ],
        compiler_params=pltpu.CompilerParams(
            use_tc_tiling_on_sc=False, needs_layout_passes=False
        ),
    )


def _build_table(input):
    x = input.transpose(2, 3, 0, 1).reshape(_HW, _N, _C)
    return _table_builder()(x)


def _terms_kernel(rois_ref, idx_ref, w_ref):
    shp = (_NROIS, _TERMS)
    t = lax.broadcasted_iota(jnp.int32, shp, 1)
    b = t // 16
    j = t - 16 * b
    ph = b // _PW
    pw = b - _PW * ph
    iy = j // 8
    ix = (j - 8 * iy) // 4
    c = j - 8 * iy - 4 * ix

    n = rois_ref[:, 0:1].astype(jnp.int32)
    sw = rois_ref[:, 1:2] * _SCALE - 0.5
    sh = rois_ref[:, 2:3] * _SCALE - 0.5
    ew = rois_ref[:, 3:4] * _SCALE - 0.5
    eh = rois_ref[:, 4:5] * _SCALE - 0.5
    bh = (eh - sh) * (1.0 / _PH)
    bw = (ew - sw) * (1.0 / _PW)

    y = sh + ph.astype(jnp.float32) * bh + (iy.astype(jnp.float32) + 0.5) * bh * (1.0 / _GRID)
    x = sw + pw.astype(jnp.float32) * bw + (ix.astype(jnp.float32) + 0.5) * bw * (1.0 / _GRID)
    valid = (y >= -1.0) & (y <= float(_H)) & (x >= -1.0) & (x <= float(_W))

    yc = jnp.maximum(y, 0.0)
    y_low = jnp.floor(yc).astype(jnp.int32)
    y_edge = y_low >= _H - 1
    y_high = jnp.where(y_edge, _H - 1, y_low + 1)
    y_low = jnp.where(y_edge, _H - 1, y_low)
    yc = jnp.where(y_edge, y_low.astype(jnp.float32), yc)
    ly = yc - y_low.astype(jnp.float32)
    hy = 1.0 - ly

    xc = jnp.maximum(x, 0.0)
    x_low = jnp.floor(xc).astype(jnp.int32)
    x_edge = x_low >= _W - 1
    x_high = jnp.where(x_edge, _W - 1, x_low + 1)
    x_low = jnp.where(x_edge, _W - 1, x_low)
    xc = jnp.where(x_edge, x_low.astype(jnp.float32), xc)
    lx = xc - x_low.astype(jnp.float32)
    hx = 1.0 - lx

    yp = jnp.where(c >= 2, y_high, y_low)
    xp = jnp.where(c % 2 == 1, x_high, x_low)
    wy = jnp.where(c >= 2, ly, hy)
    wx = jnp.where(c % 2 == 1, lx, hx)
    w = jnp.where(valid, wy * wx * (1.0 / (_GRID * _GRID)), 0.0)

    row = (n * _NBINS + b) * _HW + yp * _W + xp
    row = jnp.clip(row, 0, _VROWS - 1)
    idx_ref[...] = row
    w_ref[...] = w


def _compute_terms(rois):
    return pl.pallas_call(
        _terms_kernel,
        out_shape=(
            jax.ShapeDtypeStruct((_NROIS, _TERMS), jnp.int32),
            jax.ShapeDtypeStruct((_NROIS, _TERMS), jnp.float32),
        ),
    )(rois)


def _sc_body(table_hbm, idx_hbm, w_hbm, out_hbm, idx_v, w_v, g_v, out_v, sem):
    wid = lax.axis_index("s") * _NCORES + lax.axis_index("c")
    base = wid * _ROIS_PER_TILE
    pltpu.sync_copy(idx_hbm.at[pl.ds(base, _ROIS_PER_TILE)], idx_v)
    pltpu.sync_copy(w_hbm.at[pl.ds(base, _ROIS_PER_TILE)], w_v)
    lanes = lax.iota(jnp.int32, _LANES)
    cmask = lanes < _COUT
    cidx = lanes * _NBINS

    def per_roi(r, carry):
        copies = [
            pltpu.async_copy(
                table_hbm.at[idx_v.at[r, j]],
                g_v.at[pl.ds(j * _CHUNK, _CHUNK)],
                sem,
            )
            for j in range(_NCHUNKS)
        ]
        for cp in copies:
            cp.wait()
        for b in range(_NBINS):
            wvec = w_v[r, pl.ds(b * 16, 16)]
            acc = wvec[0] * g_v[b * 16, :]
            for j in range(1, 16):
                acc = acc + wvec[j] * g_v[b * 16 + j, :]
            plsc.store_scatter(out_v, [cidx + b], acc, mask=cmask)
        pltpu.sync_copy(out_v, out_hbm.at[base + r])
        return carry

    lax.fori_loop(0, _ROIS_PER_TILE, per_roi, 0)


@functools.cache
def _sc_gather():
    return pl.kernel(
        _sc_body,
        out_type=jax.ShapeDtypeStruct((_NROIS, _C), jnp.float32),
        mesh=plsc.VectorSubcoreMesh(
            core_axis_name="c", subcore_axis_name="s",
            num_cores=_NCORES, num_subcores=_NSUBCORES,
        ),
        scratch_types=[
            pltpu.VMEM((_ROIS_PER_TILE, _NCHUNKS, _CHUNK), jnp.int32),
            pltpu.VMEM((_ROIS_PER_TILE, _TERMS), jnp.float32),
            pltpu.VMEM((_TERMS, _LANES), jnp.float32),
            pltpu.VMEM((_C,), jnp.float32),
            pltpu.SemaphoreType.DMA,
        ],
        compiler_params=pltpu.CompilerParams(
            use_tc_tiling_on_sc=False, needs_layout_passes=False
        ),
    )


def kernel(input, rois):
    table = _build_table(input)
    idx, w = _compute_terms(rois)
    idx = idx.reshape(_NROIS, _NCHUNKS, _CHUNK)

    out = _sc_gather()(table, idx, w)
    return out.reshape(_NROIS, _COUT, _PH, _PW)


# speedup vs baseline: 1.5767x; 1.1189x over previous
import functools

import jax
import jax.numpy as jnp
from jax import lax
from jax.experimental import pallas as pl
from jax.experimental.pallas import tpu as pltpu
from jax.experimental.pallas import tpu_sc as plsc

_N, _C, _H, _W = 2, 490, 50, 50
_PH, _PW = 7, 7
_NBINS = _PH * _PW
_COUT = _C // _NBINS
_SCALE = 0.0625
_GRID = 2
_TERMS = _NBINS * _GRID * _GRID * 4
_NROIS = 512
_VROWS = _N * _NBINS * _H * _W
_LANES = 16

_NCORES, _NSUBCORES = 2, 16
_NTILES = _NCORES * _NSUBCORES
_ROIS_PER_TILE = _NROIS // _NTILES
_CHUNK = 112
_NCHUNKS = _TERMS // _CHUNK


_HW = _H * _W
_NPAIRS = _N * _NBINS
_KSTEPS = _HW // _LANES
_KREM = _HW - _KSTEPS * _LANES


_SUB = 40
_ROWW = _N * _NBINS
_PIECE_IN = _SUB * _N * _C
_PIECE_OUT = _SUB * _ROWW * _LANES


def _table_body(inp_hbm, out_hbm, in_v, out_v, idx_v, sem):
    wid = lax.axis_index("s") * _NCORES + lax.axis_index("c")
    s_lo = wid * 78 + jnp.minimum(wid, 4)
    cnt = 78 + (wid < 4).astype(jnp.int32)
    lanes = lax.iota(jnp.int32, _LANES)
    cmask = lanes < _COUT
    cvec = lanes * _NBINS
    out_hbm2 = out_hbm

    def per_piece(s0):
        pltpu.sync_copy(inp_hbm.at[pl.ds(s0, _SUB)], in_v)

        def per_pos(si, carry):
            siv = lanes * 0 + si
            for n in range(_N):
                nv = lanes * 0 + n
                for b in range(_NBINS):
                    v = plsc.load_gather(in_v, [siv, nv, cvec + b], mask=cmask)
                    out_v[si * _ROWW + n * _NBINS + b, :] = v
            return carry

        lax.fori_loop(0, _SUB, per_pos, 0)

        def idx_chunk(t, carry):
            kv = t * _LANES + lanes
            siv = kv // _ROWW
            gv = kv - siv * _ROWW
            j = t // 7
            q = t - j * 7
            idx_v[j, 0, pl.ds(q * _LANES, _LANES)] = gv * _HW + (s0 + siv)
            return carry

        lax.fori_loop(0, _SUB * _ROWW // _LANES, idx_chunk, 0)
        copies = [
            pltpu.async_copy(
                out_v.at[pl.ds(j * _CHUNK, _CHUNK)],
                out_hbm2.at[idx_v.at[j, 0]],
                sem,
            )
            for j in range(_SUB * _ROWW // _CHUNK)
        ]
        for cp in copies:
            cp.wait()

    per_piece(s_lo)
    per_piece(s_lo + cnt - _SUB)


@functools.cache
def _table_builder():
    return pl.kernel(
        _table_body,
        out_type=jax.ShapeDtypeStruct((_VROWS, _LANES), jnp.float32),
        mesh=plsc.VectorSubcoreMesh(
            core_axis_name="c", subcore_axis_name="s",
            num_cores=_NCORES, num_subcores=_NSUBCORES,
        ),
        scratch_types=[
            pltpu.VMEM((_SUB, _N, _C), jnp.float32),
            pltpu.VMEM((_SUB * _ROWW, _LANES), jnp.float32),
            pltpu.VMEM((_SUB * _ROWW // _CHUNK, 1, _CHUNK), jnp.int32),
            pltpu.SemaphoreType.DMA,
        ],
        compiler_params=pltpu.CompilerParams(
            use_tc_tiling_on_sc=False, needs_layout_passes=False
        ),
    )


def _build_table(input):
    x = input.transpose(2, 3, 0, 1).reshape(_HW, _N, _C)
    return _table_builder()(x)


def _terms_kernel(rois_ref, idx_ref, w_ref):
    shp = (_NROIS, _TERMS)
    t = lax.broadcasted_iota(jnp.int32, shp, 1)
    b = t // 16
    j = t - 16 * b
    ph = b // _PW
    pw = b - _PW * ph
    iy = j // 8
    ix = (j - 8 * iy) // 4
    c = j - 8 * iy - 4 * ix

    n = rois_ref[:, 0:1].astype(jnp.int32)
    sw = rois_ref[:, 1:2] * _SCALE - 0.5
    sh = rois_ref[:, 2:3] * _SCALE - 0.5
    ew = rois_ref[:, 3:4] * _SCALE - 0.5
    eh = rois_ref[:, 4:5] * _SCALE - 0.5
    bh = (eh - sh) * (1.0 / _PH)
    bw = (ew - sw) * (1.0 / _PW)

    y = sh + ph.astype(jnp.float32) * bh + (iy.astype(jnp.float32) + 0.5) * bh * (1.0 / _GRID)
    x = sw + pw.astype(jnp.float32) * bw + (ix.astype(jnp.float32) + 0.5) * bw * (1.0 / _GRID)
    valid = (y >= -1.0) & (y <= float(_H)) & (x >= -1.0) & (x <= float(_W))

    yc = jnp.maximum(y, 0.0)
    y_low = jnp.floor(yc).astype(jnp.int32)
    y_edge = y_low >= _H - 1
    y_high = jnp.where(y_edge, _H - 1, y_low + 1)
    y_low = jnp.where(y_edge, _H - 1, y_low)
    yc = jnp.where(y_edge, y_low.astype(jnp.float32), yc)
    ly = yc - y_low.astype(jnp.float32)
    hy = 1.0 - ly

    xc = jnp.maximum(x, 0.0)
    x_low = jnp.floor(xc).astype(jnp.int32)
    x_edge = x_low >= _W - 1
    x_high = jnp.where(x_edge, _W - 1, x_low + 1)
    x_low = jnp.where(x_edge, _W - 1, x_low)
    xc = jnp.where(x_edge, x_low.astype(jnp.float32), xc)
    lx = xc - x_low.astype(jnp.float32)
    hx = 1.0 - lx

    yp = jnp.where(c >= 2, y_high, y_low)
    xp = jnp.where(c % 2 == 1, x_high, x_low)
    wy = jnp.where(c >= 2, ly, hy)
    wx = jnp.where(c % 2 == 1, lx, hx)
    w = jnp.where(valid, wy * wx * (1.0 / (_GRID * _GRID)), 0.0)

    row = (n * _NBINS + b) * _HW + yp * _W + xp
    row = jnp.clip(row, 0, _VROWS - 1)
    idx_ref[...] = row
    w_ref[...] = w


def _compute_terms(rois):
    return pl.pallas_call(
        _terms_kernel,
        out_shape=(
            jax.ShapeDtypeStruct((_NROIS, _TERMS), jnp.int32),
            jax.ShapeDtypeStruct((_NROIS, _TERMS), jnp.float32),
        ),
    )(rois)


def _sc_body(table_hbm, idx_hbm, w_hbm, out_hbm, idx_v, w_v, g_v, out_v, sem):
    wid = lax.axis_index("s") * _NCORES + lax.axis_index("c")
    base = wid * _ROIS_PER_TILE
    pltpu.sync_copy(idx_hbm.at[pl.ds(base, _ROIS_PER_TILE)], idx_v)
    pltpu.sync_copy(w_hbm.at[pl.ds(base, _ROIS_PER_TILE)], w_v)
    lanes = lax.iota(jnp.int32, _LANES)
    cmask = lanes < _COUT
    cidx = lanes * _NBINS

    def issue(rr, par):
        for j in range(_NCHUNKS):
            pltpu.async_copy(
                table_hbm.at[idx_v.at[rr, j]],
                g_v.at[pl.ds(par * _TERMS + j * _CHUNK, _CHUNK)],
                sem,
            )

    issue(0, 0)

    def per_roi(r, carry):
        par = r & 1
        for j in range(_NCHUNKS):
            pltpu.make_async_copy(
                table_hbm.at[idx_v.at[r, j]],
                g_v.at[pl.ds(par * _TERMS + j * _CHUNK, _CHUNK)],
                sem,
            ).wait()

        @pl.when(r < _ROIS_PER_TILE - 1)
        def _():
            issue(r + 1, 1 - par)

        gb0 = par * _TERMS
        for b in range(_NBINS):
            wvec = w_v[r, pl.ds(b * 16, 16)]
            gb = gb0 + b * 16
            p = [wvec[j] * g_v[gb + j, :] for j in range(16)]
            while len(p) > 1:
                p = [p[i] + p[i + 1] for i in range(0, len(p), 2)]
            plsc.store_scatter(out_v, [cidx + b], p[0], mask=cmask)
        pltpu.sync_copy(out_v, out_hbm.at[base + r])
        return carry

    lax.fori_loop(0, _ROIS_PER_TILE, per_roi, 0)


@functools.cache
def _sc_gather():
    return pl.kernel(
        _sc_body,
        out_type=jax.ShapeDtypeStruct((_NROIS, _C), jnp.float32),
        mesh=plsc.VectorSubcoreMesh(
            core_axis_name="c", subcore_axis_name="s",
            num_cores=_NCORES, num_subcores=_NSUBCORES,
        ),
        scratch_types=[
            pltpu.VMEM((_ROIS_PER_TILE, _NCHUNKS, _CHUNK), jnp.int32),
            pltpu.VMEM((_ROIS_PER_TILE, _TERMS), jnp.float32),
            pltpu.VMEM((2 * _TERMS, _LANES), jnp.float32),
            pltpu.VMEM((_C,), jnp.float32),
            pltpu.SemaphoreType.DMA,
        ],
        compiler_params=pltpu.CompilerParams(
            use_tc_tiling_on_sc=False, needs_layout_passes=False
        ),
    )


def kernel(input, rois):
    table = _build_table(input)
    idx, w = _compute_terms(rois)
    idx = idx.reshape(_NROIS, _NCHUNKS, _CHUNK)

    out = _sc_gather()(table, idx, w)
    return out.reshape(_NROIS, _COUT, _PH, _PW)
